# Initial kernel scaffold; baseline (speedup 1.0000x reference)
import jax
import jax.numpy as jnp
from jax.experimental import pallas as pl


def kernel(z, edge_index, edge_attr, batch, emb, We0, be0, We1, be1, Wn, bn, Wr0, br0, Wr1, br1):
    raise NotImplementedError("write your pallas kernel here")


# speedup vs baseline: 1.0471x; 1.0471x over previous
import functools

import jax
import jax.numpy as jnp
from jax import lax
from jax.experimental import pallas as pl
from jax.experimental.pallas import tpu as pltpu
from jax.experimental.pallas import tpu_sc as plsc

N = 10000
E = 320000
H = 128
ED = 4
L = 4
G = 128

NC = 2
NS = 16
NW = NC * NS

ECH = 128
EW = 10240
EPAD = NW * EW
NCH_E = EW // ECH

NZW = 320
NZPAD = NW * NZW
ZCH = 80
NCH_Z = NZW // ZCH

SROWS = 10240
ZROWS_T = SROWS // NS

_MESH = plsc.VectorSubcoreMesh(
    core_axis_name="c", subcore_axis_name="s", num_cores=NC, num_subcores=NS)


def _silu(v):
    return v * jax.nn.sigmoid(v)


@functools.partial(
    pl.kernel,
    out_type=jax.ShapeDtypeStruct((NZPAD, H), jnp.float32),
    mesh=_MESH,
    scratch_types=[
        pltpu.VMEM((ZCH,), jnp.int32),
        pltpu.VMEM((ZCH, H), jnp.float32),
        pltpu.SemaphoreType.DMA,
    ],
)
def _sc_gather_emb(table_hbm, idx_hbm, out_hbm, idx_v, rows_v, sem):
    wid = lax.axis_index("s") * NC + lax.axis_index("c")
    base = wid * NZW

    def chunk(i, _):
        off = base + i * ZCH
        pltpu.sync_copy(idx_hbm.at[pl.ds(off, ZCH)], idx_v)
        pltpu.async_copy(table_hbm.at[idx_v], rows_v, sem).wait()
        pltpu.sync_copy(rows_v, out_hbm.at[pl.ds(off, ZCH)])
        return _

    lax.fori_loop(0, NCH_Z, chunk, None)


@functools.partial(
    pl.kernel,
    out_type=(
        jax.ShapeDtypeStruct((EPAD, H), jnp.float32),
        jax.ShapeDtypeStruct((EPAD, H), jnp.float32),
    ),
    mesh=_MESH,
    scratch_types=[
        pltpu.VMEM((ECH,), jnp.int32),
        pltpu.VMEM((ECH, H), jnp.float32),
        pltpu.SemaphoreType.DMA,
    ],
)
def _sc_gather_edges(p_hbm, q_hbm, src_hbm, dst_hbm, sp_hbm, sq_hbm,
                     idx_v, rows_v, sem):
    wid = lax.axis_index("s") * NC + lax.axis_index("c")
    base = wid * EW

    def chunk(i, _):
        off = base + i * ECH
        pltpu.sync_copy(src_hbm.at[pl.ds(off, ECH)], idx_v)
        pltpu.async_copy(p_hbm.at[idx_v], rows_v, sem).wait()
        pltpu.sync_copy(rows_v, sp_hbm.at[pl.ds(off, ECH)])
        pltpu.sync_copy(dst_hbm.at[pl.ds(off, ECH)], idx_v)
        pltpu.async_copy(q_hbm.at[idx_v], rows_v, sem).wait()
        pltpu.sync_copy(rows_v, sq_hbm.at[pl.ds(off, ECH)])
        return _

    lax.fori_loop(0, NCH_E, chunk, None)


@functools.partial(
    pl.kernel,
    out_type=jax.ShapeDtypeStruct((NC, SROWS, H), jnp.float32),
    mesh=_MESH,
    scratch_types=[
        pltpu.VMEM_SHARED((SROWS, H), jnp.float32),
        pltpu.VMEM((ECH,), jnp.int32),
        pltpu.VMEM((ECH, H), jnp.float32),
    ],
)
def _sc_scatter_add(msg_hbm, dst_hbm, out_hbm, acc_sh, idx_v, msg_v):
    c = lax.axis_index("c")
    s = lax.axis_index("s")
    wid = s * NC + c

    def zrow(r, _):
        for j in range(H // 16):
            msg_v[r, pl.ds(j * 16, 16)] = jnp.zeros((16,), jnp.float32)
        return _

    lax.fori_loop(0, ECH, zrow, None)

    def zchunk(i, _):
        pltpu.sync_copy(msg_v, acc_sh.at[pl.ds(s * ZROWS_T + i * ECH, ECH)])
        return _

    lax.fori_loop(0, ZROWS_T // ECH, zchunk, None)
    plsc.subcore_barrier()

    base = wid * EW

    def chunk(i, _):
        off = base + i * ECH
        pltpu.sync_copy(dst_hbm.at[pl.ds(off, ECH)], idx_v)
        pltpu.sync_copy(msg_hbm.at[pl.ds(off, ECH)], msg_v)
        pltpu.sync_copy(msg_v, acc_sh.at[idx_v], add=True)
        return _

    lax.fori_loop(0, NCH_E, chunk, None)
    plsc.subcore_barrier()

    pltpu.sync_copy(acc_sh.at[pl.ds(s * ZROWS_T, ZROWS_T)],
                    out_hbm.at[c, pl.ds(s * ZROWS_T, ZROWS_T)])


BN = 2000
BE = 2048


def _tc_pq_body(x_ref, a_ref, b_ref, p_ref, q_ref):
    x = x_ref[...]
    p_ref[...] = jnp.dot(x, a_ref[...], preferred_element_type=jnp.float32, precision=lax.Precision.HIGHEST)
    q_ref[...] = jnp.dot(x, b_ref[...], preferred_element_type=jnp.float32, precision=lax.Precision.HIGHEST)


def _tc_pq(x, wa, wb):
    return pl.pallas_call(
        _tc_pq_body,
        grid=(N // BN,),
        in_specs=[
            pl.BlockSpec((BN, H), lambda i: (i, 0)),
            pl.BlockSpec((H, H), lambda i: (0, 0)),
            pl.BlockSpec((H, H), lambda i: (0, 0)),
        ],
        out_specs=[
            pl.BlockSpec((BN, H), lambda i: (i, 0)),
            pl.BlockSpec((BN, H), lambda i: (i, 0)),
        ],
        out_shape=[
            jax.ShapeDtypeStruct((N, H), jnp.float32),
            jax.ShapeDtypeStruct((N, H), jnp.float32),
        ],
    )(x, wa, wb)


def _tc_edge_body(sp_ref, sq_ref, ea_ref, wc_ref, b0_ref, w1_ref, b1_ref,
                  out_ref):
    pre = (sp_ref[...] + sq_ref[...]
           + jnp.dot(ea_ref[...], wc_ref[...],
                     preferred_element_type=jnp.float32, precision=lax.Precision.HIGHEST) + b0_ref[...])
    h = _silu(pre)
    out_ref[...] = _silu(
        jnp.dot(h, w1_ref[...], preferred_element_type=jnp.float32, precision=lax.Precision.HIGHEST)
        + b1_ref[...])


def _tc_edge(sp, sq, ea, wc, b0, w1, b1):
    return pl.pallas_call(
        _tc_edge_body,
        grid=(EPAD // BE,),
        in_specs=[
            pl.BlockSpec((BE, H), lambda i: (i, 0)),
            pl.BlockSpec((BE, H), lambda i: (i, 0)),
            pl.BlockSpec((BE, ED), lambda i: (i, 0)),
            pl.BlockSpec((ED, H), lambda i: (0, 0)),
            pl.BlockSpec((1, H), lambda i: (0, 0)),
            pl.BlockSpec((H, H), lambda i: (0, 0)),
            pl.BlockSpec((1, H), lambda i: (0, 0)),
        ],
        out_specs=pl.BlockSpec((BE, H), lambda i: (i, 0)),
        out_shape=jax.ShapeDtypeStruct((EPAD, H), jnp.float32),
    )(sp, sq, ea, wc, b0, w1, b1)


def _tc_node_body(x_ref, p0_ref, p1_ref, wx_ref, wa_ref, b_ref, out_ref):
    agg = p0_ref[0] + p1_ref[0]
    pre = (jnp.dot(x_ref[...], wx_ref[...], preferred_element_type=jnp.float32, precision=lax.Precision.HIGHEST)
           + jnp.dot(agg, wa_ref[...], preferred_element_type=jnp.float32, precision=lax.Precision.HIGHEST)
           + b_ref[...])
    out_ref[...] = _silu(pre)


def _tc_node(x, parts, wx, wa, b):
    return pl.pallas_call(
        _tc_node_body,
        grid=(N // BN,),
        in_specs=[
            pl.BlockSpec((BN, H), lambda i: (i, 0)),
            pl.BlockSpec((1, BN, H), lambda i: (0, i, 0)),
            pl.BlockSpec((1, BN, H), lambda i: (1, i, 0)),
            pl.BlockSpec((H, H), lambda i: (0, 0)),
            pl.BlockSpec((H, H), lambda i: (0, 0)),
            pl.BlockSpec((1, H), lambda i: (0, 0)),
        ],
        out_specs=pl.BlockSpec((BN, H), lambda i: (i, 0)),
        out_shape=jax.ShapeDtypeStruct((N, H), jnp.float32),
    )(x, parts, parts, wx, wa, b)


def _tc_readout_body(x_ref, batch_ref, wr0_ref, br0_ref, wr1_ref, br1_ref,
                     out_ref):
    seg = lax.broadcasted_iota(jnp.int32, (G, 1), 0)
    mask = jnp.equal(batch_ref[...], seg).astype(jnp.float32)
    counts = jnp.sum(mask, axis=1, keepdims=True)
    gsum = jnp.dot(mask, x_ref[...], preferred_element_type=jnp.float32, precision=lax.Precision.HIGHEST)
    g = gsum / jnp.maximum(counts, 1.0)
    hr = _silu(jnp.dot(g, wr0_ref[...], preferred_element_type=jnp.float32, precision=lax.Precision.HIGHEST)
               + br0_ref[...])
    out_ref[...] = (jnp.dot(hr, wr1_ref[...],
                            preferred_element_type=jnp.float32, precision=lax.Precision.HIGHEST) + br1_ref[...])


def _tc_readout(x, batch2d, wr0, br0, wr1, br1):
    return pl.pallas_call(
        _tc_readout_body,
        grid=(1,),
        in_specs=[
            pl.BlockSpec((N, H), lambda i: (0, 0)),
            pl.BlockSpec((1, N), lambda i: (0, 0)),
            pl.BlockSpec((H, H), lambda i: (0, 0)),
            pl.BlockSpec((1, H), lambda i: (0, 0)),
            pl.BlockSpec((H, 1), lambda i: (0, 0)),
            pl.BlockSpec((1, 1), lambda i: (0, 0)),
        ],
        out_specs=pl.BlockSpec((G, 1), lambda i: (0, 0)),
        out_shape=jax.ShapeDtypeStruct((G, 1), jnp.float32),
    )(x, batch2d, wr0, br0, wr1, br1)


def kernel(z, edge_index, edge_attr, batch, emb, We0, be0, We1, be1,
           Wn, bn, Wr0, br0, Wr1, br1):
    src = edge_index[0].astype(jnp.int32)
    dst = edge_index[1].astype(jnp.int32)

    srcp = jnp.concatenate([src, jnp.zeros((EPAD - E,), jnp.int32)])
    dstp_g = jnp.concatenate([dst, jnp.zeros((EPAD - E,), jnp.int32)])
    dstp_s = jnp.concatenate([dst, jnp.full((EPAD - E,), N, jnp.int32)])
    eap = jnp.concatenate(
        [edge_attr, jnp.zeros((EPAD - E, ED), jnp.float32)], axis=0)
    zp = jnp.concatenate(
        [z.astype(jnp.int32), jnp.zeros((NZPAD - N,), jnp.int32)])

    x = _sc_gather_emb(emb, zp)[:N]

    for l in range(L):
        wa = We0[l, :H]
        wb = We0[l, H:2 * H]
        wc = We0[l, 2 * H:]
        p, q = _tc_pq(x, wa, wb)
        sp, sq = _sc_gather_edges(p, q, srcp, dstp_g)
        msg = _tc_edge(sp, sq, eap, wc, be0[l].reshape(1, H),
                       We1[l], be1[l].reshape(1, H))
        parts = _sc_scatter_add(msg, dstp_s)
        x = _tc_node(x, parts, Wn[l, :H], Wn[l, H:], bn[l].reshape(1, H))

    pred = _tc_readout(x, batch.astype(jnp.int32).reshape(1, N),
                       Wr0, br0.reshape(1, H), Wr1, br1.reshape(1, 1))
    return pred.reshape(G)


# speedup vs baseline: 1.6041x; 1.5319x over previous
import functools

import jax
import jax.numpy as jnp
from jax import lax
from jax.experimental import pallas as pl
from jax.experimental.pallas import tpu as pltpu
from jax.experimental.pallas import tpu_sc as plsc

N = 10000
E = 320000
H = 128
ED = 4
L = 4
G = 128

NC = 2
NS = 16
NW = NC * NS

ECH = 128
EW = 10240
EPAD = NW * EW
NCH_E = EW // ECH

NZW = 320
NZPAD = NW * NZW
ZCH = 80
NCH_Z = NZW // ZCH

SROWS = 10240
ZROWS_T = SROWS // NS

_MESH = plsc.VectorSubcoreMesh(
    core_axis_name="c", subcore_axis_name="s", num_cores=NC, num_subcores=NS)


def _silu(v):
    return v * jax.nn.sigmoid(v)


@functools.partial(
    pl.kernel,
    out_type=jax.ShapeDtypeStruct((NZPAD, H), jnp.float32),
    mesh=_MESH,
    scratch_types=[
        pltpu.VMEM((ZCH,), jnp.int32),
        pltpu.VMEM((ZCH, H), jnp.float32),
        pltpu.SemaphoreType.DMA,
    ],
)
def _sc_gather_emb(table_hbm, idx_hbm, out_hbm, idx_v, rows_v, sem):
    wid = lax.axis_index("s") * NC + lax.axis_index("c")
    base = wid * NZW

    def chunk(i, _):
        off = base + i * ZCH
        pltpu.sync_copy(idx_hbm.at[pl.ds(off, ZCH)], idx_v)
        pltpu.async_copy(table_hbm.at[idx_v], rows_v, sem).wait()
        pltpu.sync_copy(rows_v, out_hbm.at[pl.ds(off, ZCH)])
        return _

    lax.fori_loop(0, NCH_Z, chunk, None)


@functools.partial(
    pl.kernel,
    out_type=(
        jax.ShapeDtypeStruct((EPAD, H), jnp.float32),
        jax.ShapeDtypeStruct((EPAD, H), jnp.float32),
    ),
    mesh=_MESH,
    scratch_types=[
        pltpu.VMEM((NCH_E, ECH), jnp.int32),
        pltpu.VMEM((NCH_E, ECH), jnp.int32),
        pltpu.VMEM((ECH, H), jnp.float32),
        pltpu.VMEM((ECH, H), jnp.float32),
        pltpu.VMEM((ECH, H), jnp.float32),
        pltpu.VMEM((ECH, H), jnp.float32),
        pltpu.SemaphoreType.DMA,
        pltpu.SemaphoreType.DMA,
        pltpu.SemaphoreType.DMA,
        pltpu.SemaphoreType.DMA,
    ],
)
def _sc_gather_edges(p_hbm, q_hbm, src_hbm, dst_hbm, sp_hbm, sq_hbm,
                     idxs_v, idxd_v, bp0, bp1, bq0, bq1,
                     sg0, sg1, sw0, sw1):
    wid = lax.axis_index("s") * NC + lax.axis_index("c")
    base = wid * EW
    bufp = (bp0, bp1)
    bufq = (bq0, bq1)
    semg = (sg0, sg1)
    semw = (sw0, sw1)

    pltpu.sync_copy(src_hbm.at[wid], idxs_v)
    pltpu.sync_copy(dst_hbm.at[wid], idxd_v)

    def issue_gather(i, b):
        pltpu.async_copy(p_hbm.at[idxs_v.at[i]], bufp[b], semg[b])
        pltpu.async_copy(q_hbm.at[idxd_v.at[i]], bufq[b], semg[b])

    def wait_gather(b):
        pltpu.make_async_copy(p_hbm.at[idxs_v.at[0]], bufp[b], semg[b]).wait()
        pltpu.make_async_copy(q_hbm.at[idxd_v.at[0]], bufq[b], semg[b]).wait()

    def issue_write(i, b):
        off = base + i * ECH
        pltpu.async_copy(bufp[b], sp_hbm.at[pl.ds(off, ECH)], semw[b])
        pltpu.async_copy(bufq[b], sq_hbm.at[pl.ds(off, ECH)], semw[b])

    def wait_write(b):
        pltpu.make_async_copy(bufp[b], sp_hbm.at[pl.ds(0, ECH)], semw[b]).wait()
        pltpu.make_async_copy(bufq[b], sq_hbm.at[pl.ds(0, ECH)], semw[b]).wait()

    issue_gather(0, 0)
    issue_gather(1, 1)

    def outer(k, _):
        for b in range(2):
            i = k * 2 + b
            wait_gather(b)
            issue_write(i, b)

            @pl.when(i + 2 < NCH_E)
            def _():
                wait_write(b)
                issue_gather(i + 2, b)

        return _

    lax.fori_loop(0, NCH_E // 2, outer, None)
    wait_write(0)
    wait_write(1)


@functools.partial(
    pl.kernel,
    out_type=jax.ShapeDtypeStruct((NC, SROWS, H), jnp.float32),
    mesh=_MESH,
    scratch_types=[
        pltpu.VMEM_SHARED((SROWS, H), jnp.float32),
        pltpu.VMEM((NCH_E, ECH), jnp.int32),
        pltpu.VMEM((ECH, H), jnp.float32),
        pltpu.VMEM((ECH, H), jnp.float32),
        pltpu.SemaphoreType.DMA,
        pltpu.SemaphoreType.DMA,
        pltpu.SemaphoreType.DMA,
        pltpu.SemaphoreType.DMA,
    ],
)
def _sc_scatter_add(msg_hbm, dst_hbm, out_hbm, acc_sh, idx_v, mb0, mb1,
                    sm0, sm1, sa0, sa1):
    c = lax.axis_index("c")
    s = lax.axis_index("s")
    wid = s * NC + c
    bufm = (mb0, mb1)
    semm = (sm0, sm1)
    sema = (sa0, sa1)

    def zrow(r, _):
        for j in range(H // 16):
            mb0[r, pl.ds(j * 16, 16)] = jnp.zeros((16,), jnp.float32)
        return _

    lax.fori_loop(0, ECH, zrow, None)

    def zchunk(i, _):
        pltpu.sync_copy(mb0, acc_sh.at[pl.ds(s * ZROWS_T + i * ECH, ECH)])
        return _

    lax.fori_loop(0, ZROWS_T // ECH, zchunk, None)
    pltpu.sync_copy(dst_hbm.at[wid], idx_v)
    plsc.subcore_barrier()

    base = wid * EW

    def issue_load(i, b):
        pltpu.async_copy(msg_hbm.at[pl.ds(base + i * ECH, ECH)], bufm[b],
                         semm[b])

    def wait_load(b):
        pltpu.make_async_copy(msg_hbm.at[pl.ds(0, ECH)], bufm[b],
                              semm[b]).wait()

    def issue_scatter(i, b):
        pltpu.async_copy(bufm[b], acc_sh.at[idx_v.at[i]], sema[b], add=True)

    def wait_scatter(b):
        pltpu.make_async_copy(bufm[b], acc_sh.at[idx_v.at[0]], sema[b]).wait()

    issue_load(0, 0)
    issue_load(1, 1)

    def outer(k, _):
        for b in range(2):
            i = k * 2 + b
            wait_load(b)
            issue_scatter(i, b)

            @pl.when(i + 2 < NCH_E)
            def _():
                wait_scatter(b)
                issue_load(i + 2, b)

        return _

    lax.fori_loop(0, NCH_E // 2, outer, None)
    wait_scatter(0)
    wait_scatter(1)
    plsc.subcore_barrier()

    pltpu.sync_copy(acc_sh.at[pl.ds(s * ZROWS_T, ZROWS_T)],
                    out_hbm.at[c, pl.ds(s * ZROWS_T, ZROWS_T)])


BN = 2000
BE = 2048


def _tc_pq_body(x_ref, a_ref, b_ref, p_ref, q_ref):
    x = x_ref[...]
    p_ref[...] = jnp.dot(x, a_ref[...], preferred_element_type=jnp.float32, precision=lax.Precision.HIGHEST)
    q_ref[...] = jnp.dot(x, b_ref[...], preferred_element_type=jnp.float32, precision=lax.Precision.HIGHEST)


def _tc_pq(x, wa, wb):
    return pl.pallas_call(
        _tc_pq_body,
        grid=(N // BN,),
        in_specs=[
            pl.BlockSpec((BN, H), lambda i: (i, 0)),
            pl.BlockSpec((H, H), lambda i: (0, 0)),
            pl.BlockSpec((H, H), lambda i: (0, 0)),
        ],
        out_specs=[
            pl.BlockSpec((BN, H), lambda i: (i, 0)),
            pl.BlockSpec((BN, H), lambda i: (i, 0)),
        ],
        out_shape=[
            jax.ShapeDtypeStruct((N, H), jnp.float32),
            jax.ShapeDtypeStruct((N, H), jnp.float32),
        ],
    )(x, wa, wb)


def _tc_edge_body(sp_ref, sq_ref, ea_ref, wc_ref, b0_ref, w1_ref, b1_ref,
                  out_ref):
    pre = (sp_ref[...] + sq_ref[...]
           + jnp.dot(ea_ref[...], wc_ref[...],
                     preferred_element_type=jnp.float32, precision=lax.Precision.HIGHEST) + b0_ref[...])
    h = _silu(pre)
    out_ref[...] = _silu(
        jnp.dot(h, w1_ref[...], preferred_element_type=jnp.float32, precision=lax.Precision.HIGHEST)
        + b1_ref[...])


def _tc_edge(sp, sq, ea, wc, b0, w1, b1):
    return pl.pallas_call(
        _tc_edge_body,
        grid=(EPAD // BE,),
        in_specs=[
            pl.BlockSpec((BE, H), lambda i: (i, 0)),
            pl.BlockSpec((BE, H), lambda i: (i, 0)),
            pl.BlockSpec((BE, ED), lambda i: (i, 0)),
            pl.BlockSpec((ED, H), lambda i: (0, 0)),
            pl.BlockSpec((1, H), lambda i: (0, 0)),
            pl.BlockSpec((H, H), lambda i: (0, 0)),
            pl.BlockSpec((1, H), lambda i: (0, 0)),
        ],
        out_specs=pl.BlockSpec((BE, H), lambda i: (i, 0)),
        out_shape=jax.ShapeDtypeStruct((EPAD, H), jnp.float32),
    )(sp, sq, ea, wc, b0, w1, b1)


def _tc_node_body(x_ref, p0_ref, p1_ref, wx_ref, wa_ref, b_ref, out_ref):
    agg = p0_ref[0] + p1_ref[0]
    pre = (jnp.dot(x_ref[...], wx_ref[...], preferred_element_type=jnp.float32, precision=lax.Precision.HIGHEST)
           + jnp.dot(agg, wa_ref[...], preferred_element_type=jnp.float32, precision=lax.Precision.HIGHEST)
           + b_ref[...])
    out_ref[...] = _silu(pre)


def _tc_node(x, parts, wx, wa, b):
    return pl.pallas_call(
        _tc_node_body,
        grid=(N // BN,),
        in_specs=[
            pl.BlockSpec((BN, H), lambda i: (i, 0)),
            pl.BlockSpec((1, BN, H), lambda i: (0, i, 0)),
            pl.BlockSpec((1, BN, H), lambda i: (1, i, 0)),
            pl.BlockSpec((H, H), lambda i: (0, 0)),
            pl.BlockSpec((H, H), lambda i: (0, 0)),
            pl.BlockSpec((1, H), lambda i: (0, 0)),
        ],
        out_specs=pl.BlockSpec((BN, H), lambda i: (i, 0)),
        out_shape=jax.ShapeDtypeStruct((N, H), jnp.float32),
    )(x, parts, parts, wx, wa, b)


def _tc_readout_body(x_ref, batch_ref, wr0_ref, br0_ref, wr1_ref, br1_ref,
                     out_ref):
    seg = lax.broadcasted_iota(jnp.int32, (G, 1), 0)
    mask = jnp.equal(batch_ref[...], seg).astype(jnp.float32)
    counts = jnp.sum(mask, axis=1, keepdims=True)
    gsum = jnp.dot(mask, x_ref[...], preferred_element_type=jnp.float32, precision=lax.Precision.HIGHEST)
    g = gsum / jnp.maximum(counts, 1.0)
    hr = _silu(jnp.dot(g, wr0_ref[...], preferred_element_type=jnp.float32, precision=lax.Precision.HIGHEST)
               + br0_ref[...])
    out_ref[...] = (jnp.dot(hr, wr1_ref[...],
                            preferred_element_type=jnp.float32, precision=lax.Precision.HIGHEST) + br1_ref[...])


def _tc_readout(x, batch2d, wr0, br0, wr1, br1):
    return pl.pallas_call(
        _tc_readout_body,
        grid=(1,),
        in_specs=[
            pl.BlockSpec((N, H), lambda i: (0, 0)),
            pl.BlockSpec((1, N), lambda i: (0, 0)),
            pl.BlockSpec((H, H), lambda i: (0, 0)),
            pl.BlockSpec((1, H), lambda i: (0, 0)),
            pl.BlockSpec((H, 1), lambda i: (0, 0)),
            pl.BlockSpec((1, 1), lambda i: (0, 0)),
        ],
        out_specs=pl.BlockSpec((G, 1), lambda i: (0, 0)),
        out_shape=jax.ShapeDtypeStruct((G, 1), jnp.float32),
    )(x, batch2d, wr0, br0, wr1, br1)


def kernel(z, edge_index, edge_attr, batch, emb, We0, be0, We1, be1,
           Wn, bn, Wr0, br0, Wr1, br1):
    src = edge_index[0].astype(jnp.int32)
    dst = edge_index[1].astype(jnp.int32)

    srcp = jnp.concatenate(
        [src, jnp.zeros((EPAD - E,), jnp.int32)]).reshape(NW, NCH_E, ECH)
    dstp_g = jnp.concatenate(
        [dst, jnp.zeros((EPAD - E,), jnp.int32)]).reshape(NW, NCH_E, ECH)
    dstp_s = jnp.concatenate(
        [dst, jnp.full((EPAD - E,), N, jnp.int32)]).reshape(NW, NCH_E, ECH)
    eap = jnp.concatenate(
        [edge_attr, jnp.zeros((EPAD - E, ED), jnp.float32)], axis=0)
    zp = jnp.concatenate(
        [z.astype(jnp.int32), jnp.zeros((NZPAD - N,), jnp.int32)])

    x = _sc_gather_emb(emb, zp)[:N]

    for l in range(L):
        wa = We0[l, :H]
        wb = We0[l, H:2 * H]
        wc = We0[l, 2 * H:]
        p, q = _tc_pq(x, wa, wb)
        sp, sq = _sc_gather_edges(p, q, srcp, dstp_g)
        msg = _tc_edge(sp, sq, eap, wc, be0[l].reshape(1, H),
                       We1[l], be1[l].reshape(1, H))
        parts = _sc_scatter_add(msg, dstp_s)
        x = _tc_node(x, parts, Wn[l, :H], Wn[l, H:], bn[l].reshape(1, H))

    pred = _tc_readout(x, batch.astype(jnp.int32).reshape(1, N),
                       Wr0, br0.reshape(1, H), Wr1, br1.reshape(1, 1))
    return pred.reshape(G)


# speedup vs baseline: 1.7476x; 1.0895x over previous
import functools

import jax
import jax.numpy as jnp
from jax import lax
from jax.experimental import pallas as pl
from jax.experimental.pallas import tpu as pltpu
from jax.experimental.pallas import tpu_sc as plsc

N = 10000
E = 320000
H = 128
ED = 4
L = 4
G = 128

NC = 2
NS = 16
NW = NC * NS

ECH = 128
EW = 10240
EPAD = NW * EW
NCH_E = EW // ECH

NZW = 320
NZPAD = NW * NZW
ZCH = 80
NCH_Z = NZW // ZCH

SROWS = 10240
ZROWS_T = SROWS // NS

_MESH = plsc.VectorSubcoreMesh(
    core_axis_name="c", subcore_axis_name="s", num_cores=NC, num_subcores=NS)


def _silu(v):
    return v * jax.nn.sigmoid(v)


@functools.partial(
    pl.kernel,
    out_type=jax.ShapeDtypeStruct((NZPAD, H), jnp.float32),
    mesh=_MESH,
    scratch_types=[
        pltpu.VMEM((ZCH,), jnp.int32),
        pltpu.VMEM((ZCH, H), jnp.float32),
        pltpu.SemaphoreType.DMA,
    ],
)
def _sc_gather_emb(table_hbm, idx_hbm, out_hbm, idx_v, rows_v, sem):
    wid = lax.axis_index("s") * NC + lax.axis_index("c")
    base = wid * NZW

    def chunk(i, _):
        off = base + i * ZCH
        pltpu.sync_copy(idx_hbm.at[pl.ds(off, ZCH)], idx_v)
        pltpu.async_copy(table_hbm.at[idx_v], rows_v, sem).wait()
        pltpu.sync_copy(rows_v, out_hbm.at[pl.ds(off, ZCH)])
        return _

    lax.fori_loop(0, NCH_Z, chunk, None)


@functools.partial(
    pl.kernel,
    out_type=jax.ShapeDtypeStruct((EPAD, H), jnp.float32),
    mesh=_MESH,
    scratch_types=[
        pltpu.VMEM((NCH_E, ECH), jnp.int32),
        pltpu.VMEM((NCH_E, ECH), jnp.int32),
        pltpu.VMEM((ECH, H), jnp.float32),
        pltpu.VMEM((ECH, H), jnp.float32),
        pltpu.VMEM((ECH, H), jnp.float32),
        pltpu.VMEM((ECH, H), jnp.float32),
        pltpu.SemaphoreType.DMA,
        pltpu.SemaphoreType.DMA,
        pltpu.SemaphoreType.DMA,
        pltpu.SemaphoreType.DMA,
        pltpu.SemaphoreType.DMA,
        pltpu.SemaphoreType.DMA,
    ],
)
def _sc_gather_edges(p_hbm, q_hbm, src_hbm, dst_hbm, s_hbm,
                     idxs_v, idxd_v, bp0, bp1, bq0, bq1,
                     sgp0, sgp1, sgq0, sgq1, sw0, sw1):
    wid = lax.axis_index("s") * NC + lax.axis_index("c")
    base = wid * EW
    bufp = (bp0, bp1)
    bufq = (bq0, bq1)
    semgp = (sgp0, sgp1)
    semgq = (sgq0, sgq1)
    semw = (sw0, sw1)

    pltpu.sync_copy(src_hbm.at[wid], idxs_v)
    pltpu.sync_copy(dst_hbm.at[wid], idxd_v)

    def issue_gather(i, b):
        pltpu.async_copy(p_hbm.at[idxs_v.at[i]], bufp[b], semgp[b])
        pltpu.async_copy(q_hbm.at[idxd_v.at[i]], bufq[b], semgq[b])

    def wait_gather(b):
        pltpu.make_async_copy(p_hbm.at[idxs_v.at[0]], bufp[b], semgp[b]).wait()
        pltpu.make_async_copy(q_hbm.at[idxd_v.at[0]], bufq[b], semgq[b]).wait()

    def issue_write(i, b):
        pltpu.async_copy(bufp[b], s_hbm.at[pl.ds(base + i * ECH, ECH)],
                         semw[b])

    def wait_write(b):
        pltpu.make_async_copy(bufp[b], s_hbm.at[pl.ds(0, ECH)], semw[b]).wait()

    issue_gather(0, 0)
    issue_gather(1, 1)

    def outer(k, _):
        for b in range(2):
            i = k * 2 + b
            wait_gather(b)
            p, q = bufp[b], bufq[b]

            def add_row(r, _):
                for j in range(H // 16):
                    sl = pl.ds(j * 16, 16)
                    p[r, sl] = p[r, sl] + q[r, sl]
                return _

            lax.fori_loop(0, ECH, add_row, None)
            issue_write(i, b)

            @pl.when(i + 2 < NCH_E)
            def _():
                wait_write(b)
                issue_gather(i + 2, b)

        return _

    lax.fori_loop(0, NCH_E // 2, outer, None)
    wait_write(0)
    wait_write(1)


@functools.partial(
    pl.kernel,
    out_type=jax.ShapeDtypeStruct((NC, SROWS, H), jnp.float32),
    mesh=_MESH,
    scratch_types=[
        pltpu.VMEM_SHARED((SROWS, H), jnp.float32),
        pltpu.VMEM((NCH_E, ECH), jnp.int32),
        pltpu.VMEM((ECH, H), jnp.float32),
        pltpu.VMEM((ECH, H), jnp.float32),
        pltpu.SemaphoreType.DMA,
        pltpu.SemaphoreType.DMA,
        pltpu.SemaphoreType.DMA,
        pltpu.SemaphoreType.DMA,
    ],
)
def _sc_scatter_add(msg_hbm, dst_hbm, out_hbm, acc_sh, idx_v, mb0, mb1,
                    sm0, sm1, sa0, sa1):
    c = lax.axis_index("c")
    s = lax.axis_index("s")
    wid = s * NC + c
    bufm = (mb0, mb1)
    semm = (sm0, sm1)
    sema = (sa0, sa1)

    def zrow(r, _):
        for j in range(H // 16):
            mb0[r, pl.ds(j * 16, 16)] = jnp.zeros((16,), jnp.float32)
        return _

    lax.fori_loop(0, ECH, zrow, None)

    def zchunk(i, _):
        pltpu.sync_copy(mb0, acc_sh.at[pl.ds(s * ZROWS_T + i * ECH, ECH)])
        return _

    lax.fori_loop(0, ZROWS_T // ECH, zchunk, None)
    pltpu.sync_copy(dst_hbm.at[wid], idx_v)
    plsc.subcore_barrier()

    base = wid * EW

    def issue_load(i, b):
        pltpu.async_copy(msg_hbm.at[pl.ds(base + i * ECH, ECH)], bufm[b],
                         semm[b])

    def wait_load(b):
        pltpu.make_async_copy(msg_hbm.at[pl.ds(0, ECH)], bufm[b],
                              semm[b]).wait()

    def issue_scatter(i, b):
        pltpu.async_copy(bufm[b], acc_sh.at[idx_v.at[i]], sema[b], add=True)

    def wait_scatter(b):
        pltpu.make_async_copy(bufm[b], acc_sh.at[idx_v.at[0]], sema[b]).wait()

    issue_load(0, 0)
    issue_load(1, 1)

    def outer(k, _):
        for b in range(2):
            i = k * 2 + b
            wait_load(b)
            issue_scatter(i, b)

            @pl.when(i + 2 < NCH_E)
            def _():
                wait_scatter(b)
                issue_load(i + 2, b)

        return _

    lax.fori_loop(0, NCH_E // 2, outer, None)
    wait_scatter(0)
    wait_scatter(1)
    plsc.subcore_barrier()

    pltpu.sync_copy(acc_sh.at[pl.ds(s * ZROWS_T, ZROWS_T)],
                    out_hbm.at[c, pl.ds(s * ZROWS_T, ZROWS_T)])


BN = 2000
BE = 2048


def _tc_pq_body(x_ref, a_ref, b_ref, p_ref, q_ref):
    x = x_ref[...]
    p_ref[...] = jnp.dot(x, a_ref[...], preferred_element_type=jnp.float32, precision=lax.Precision.HIGHEST)
    q_ref[...] = jnp.dot(x, b_ref[...], preferred_element_type=jnp.float32, precision=lax.Precision.HIGHEST)


def _tc_pq(x, wa, wb):
    return pl.pallas_call(
        _tc_pq_body,
        grid=(N // BN,),
        in_specs=[
            pl.BlockSpec((BN, H), lambda i: (i, 0)),
            pl.BlockSpec((H, H), lambda i: (0, 0)),
            pl.BlockSpec((H, H), lambda i: (0, 0)),
        ],
        out_specs=[
            pl.BlockSpec((BN, H), lambda i: (i, 0)),
            pl.BlockSpec((BN, H), lambda i: (i, 0)),
        ],
        out_shape=[
            jax.ShapeDtypeStruct((N, H), jnp.float32),
            jax.ShapeDtypeStruct((N, H), jnp.float32),
        ],
    )(x, wa, wb)


def _tc_edge_body(s_ref, ea_ref, wc_ref, b0_ref, w1_ref, b1_ref,
                  out_ref):
    pre = (s_ref[...]
           + jnp.dot(ea_ref[...], wc_ref[...],
                     preferred_element_type=jnp.float32, precision=lax.Precision.HIGHEST) + b0_ref[...])
    h = _silu(pre)
    out_ref[...] = _silu(
        jnp.dot(h, w1_ref[...], preferred_element_type=jnp.float32, precision=lax.Precision.HIGHEST)
        + b1_ref[...])


def _tc_edge(s, ea, wc, b0, w1, b1):
    return pl.pallas_call(
        _tc_edge_body,
        grid=(EPAD // BE,),
        in_specs=[
            pl.BlockSpec((BE, H), lambda i: (i, 0)),
            pl.BlockSpec((BE, ED), lambda i: (i, 0)),
            pl.BlockSpec((ED, H), lambda i: (0, 0)),
            pl.BlockSpec((1, H), lambda i: (0, 0)),
            pl.BlockSpec((H, H), lambda i: (0, 0)),
            pl.BlockSpec((1, H), lambda i: (0, 0)),
        ],
        out_specs=pl.BlockSpec((BE, H), lambda i: (i, 0)),
        out_shape=jax.ShapeDtypeStruct((EPAD, H), jnp.float32),
    )(s, ea, wc, b0, w1, b1)


def _tc_node_body(x_ref, p0_ref, p1_ref, wx_ref, wa_ref, b_ref, out_ref):
    agg = p0_ref[0] + p1_ref[0]
    pre = (jnp.dot(x_ref[...], wx_ref[...], preferred_element_type=jnp.float32, precision=lax.Precision.HIGHEST)
           + jnp.dot(agg, wa_ref[...], preferred_element_type=jnp.float32, precision=lax.Precision.HIGHEST)
           + b_ref[...])
    out_ref[...] = _silu(pre)


def _tc_node(x, parts, wx, wa, b):
    return pl.pallas_call(
        _tc_node_body,
        grid=(N // BN,),
        in_specs=[
            pl.BlockSpec((BN, H), lambda i: (i, 0)),
            pl.BlockSpec((1, BN, H), lambda i: (0, i, 0)),
            pl.BlockSpec((1, BN, H), lambda i: (1, i, 0)),
            pl.BlockSpec((H, H), lambda i: (0, 0)),
            pl.BlockSpec((H, H), lambda i: (0, 0)),
            pl.BlockSpec((1, H), lambda i: (0, 0)),
        ],
        out_specs=pl.BlockSpec((BN, H), lambda i: (i, 0)),
        out_shape=jax.ShapeDtypeStruct((N, H), jnp.float32),
    )(x, parts, parts, wx, wa, b)


def _tc_readout_body(x_ref, batch_ref, wr0_ref, br0_ref, wr1_ref, br1_ref,
                     out_ref):
    seg = lax.broadcasted_iota(jnp.int32, (G, 1), 0)
    mask = jnp.equal(batch_ref[...], seg).astype(jnp.float32)
    counts = jnp.sum(mask, axis=1, keepdims=True)
    gsum = jnp.dot(mask, x_ref[...], preferred_element_type=jnp.float32, precision=lax.Precision.HIGHEST)
    g = gsum / jnp.maximum(counts, 1.0)
    hr = _silu(jnp.dot(g, wr0_ref[...], preferred_element_type=jnp.float32, precision=lax.Precision.HIGHEST)
               + br0_ref[...])
    out_ref[...] = (jnp.dot(hr, wr1_ref[...],
                            preferred_element_type=jnp.float32, precision=lax.Precision.HIGHEST) + br1_ref[...])


def _tc_readout(x, batch2d, wr0, br0, wr1, br1):
    return pl.pallas_call(
        _tc_readout_body,
        grid=(1,),
        in_specs=[
            pl.BlockSpec((N, H), lambda i: (0, 0)),
            pl.BlockSpec((1, N), lambda i: (0, 0)),
            pl.BlockSpec((H, H), lambda i: (0, 0)),
            pl.BlockSpec((1, H), lambda i: (0, 0)),
            pl.BlockSpec((H, 1), lambda i: (0, 0)),
            pl.BlockSpec((1, 1), lambda i: (0, 0)),
        ],
        out_specs=pl.BlockSpec((G, 1), lambda i: (0, 0)),
        out_shape=jax.ShapeDtypeStruct((G, 1), jnp.float32),
    )(x, batch2d, wr0, br0, wr1, br1)


def kernel(z, edge_index, edge_attr, batch, emb, We0, be0, We1, be1,
           Wn, bn, Wr0, br0, Wr1, br1):
    src = edge_index[0].astype(jnp.int32)
    dst = edge_index[1].astype(jnp.int32)

    srcp = jnp.concatenate(
        [src, jnp.zeros((EPAD - E,), jnp.int32)]).reshape(NW, NCH_E, ECH)
    dstp_g = jnp.concatenate(
        [dst, jnp.zeros((EPAD - E,), jnp.int32)]).reshape(NW, NCH_E, ECH)
    dstp_s = jnp.concatenate(
        [dst, jnp.full((EPAD - E,), N, jnp.int32)]).reshape(NW, NCH_E, ECH)
    eap = jnp.concatenate(
        [edge_attr, jnp.zeros((EPAD - E, ED), jnp.float32)], axis=0)
    zp = jnp.concatenate(
        [z.astype(jnp.int32), jnp.zeros((NZPAD - N,), jnp.int32)])

    x = _sc_gather_emb(emb, zp)[:N]

    for l in range(L):
        wa = We0[l, :H]
        wb = We0[l, H:2 * H]
        wc = We0[l, 2 * H:]
        p, q = _tc_pq(x, wa, wb)
        s = _sc_gather_edges(p, q, srcp, dstp_g)
        msg = _tc_edge(s, eap, wc, be0[l].reshape(1, H),
                       We1[l], be1[l].reshape(1, H))
        parts = _sc_scatter_add(msg, dstp_s)
        x = _tc_node(x, parts, Wn[l, :H], Wn[l, H:], bn[l].reshape(1, H))

    pred = _tc_readout(x, batch.astype(jnp.int32).reshape(1, N),
                       Wr0, br0.reshape(1, H), Wr1, br1.reshape(1, 1))
    return pred.reshape(G)


# speedup vs baseline: 2.2067x; 1.2627x over previous
import functools

import jax
import jax.numpy as jnp
from jax import lax
from jax.experimental import pallas as pl
from jax.experimental.pallas import tpu as pltpu
from jax.experimental.pallas import tpu_sc as plsc

N = 10000
E = 320000
H = 128
ED = 4
L = 4
G = 128

NC = 2
NS = 16
NW = NC * NS

ECH = 128
EW = 10240
EPAD = NW * EW
NCH_E = EW // ECH

NZW = 320
NZPAD = NW * NZW
ZCH = 80
NCH_Z = NZW // ZCH

SROWS = 10240
ZROWS_T = SROWS // NS

_MESH = plsc.VectorSubcoreMesh(
    core_axis_name="c", subcore_axis_name="s", num_cores=NC, num_subcores=NS)


def _silu(v):
    return v * jax.nn.sigmoid(v)


@functools.partial(
    pl.kernel,
    out_type=jax.ShapeDtypeStruct((NZPAD, H), jnp.float32),
    mesh=_MESH,
    scratch_types=[
        pltpu.VMEM((ZCH,), jnp.int32),
        pltpu.VMEM((ZCH, H), jnp.float32),
        pltpu.SemaphoreType.DMA,
    ],
)
def _sc_gather_emb(table_hbm, idx_hbm, out_hbm, idx_v, rows_v, sem):
    wid = lax.axis_index("s") * NC + lax.axis_index("c")
    base = wid * NZW

    def chunk(i, _):
        off = base + i * ZCH
        pltpu.sync_copy(idx_hbm.at[pl.ds(off, ZCH)], idx_v)
        pltpu.async_copy(table_hbm.at[idx_v], rows_v, sem).wait()
        pltpu.sync_copy(rows_v, out_hbm.at[pl.ds(off, ZCH)])
        return _

    lax.fori_loop(0, NCH_Z, chunk, None)


@functools.partial(
    pl.kernel,
    out_type=jax.ShapeDtypeStruct((EPAD, H), jnp.float32),
    mesh=_MESH,
    scratch_types=[
        pltpu.VMEM((NCH_E, ECH), jnp.int32),
        pltpu.VMEM((NCH_E, ECH), jnp.int32),
        pltpu.VMEM((ECH, H), jnp.float32),
        pltpu.VMEM((ECH, H), jnp.float32),
        pltpu.VMEM((ECH, H), jnp.float32),
        pltpu.VMEM((ECH, H), jnp.float32),
        pltpu.SemaphoreType.DMA,
        pltpu.SemaphoreType.DMA,
        pltpu.SemaphoreType.DMA,
        pltpu.SemaphoreType.DMA,
        pltpu.SemaphoreType.DMA,
        pltpu.SemaphoreType.DMA,
    ],
)
def _sc_gather_edges(p_hbm, q_hbm, src_hbm, dst_hbm, s_hbm,
                     idxs_v, idxd_v, bp0, bp1, bq0, bq1,
                     sgp0, sgp1, sgq0, sgq1, sw0, sw1):
    wid = lax.axis_index("s") * NC + lax.axis_index("c")
    base = wid * EW
    bufp = (bp0, bp1)
    bufq = (bq0, bq1)
    semgp = (sgp0, sgp1)
    semgq = (sgq0, sgq1)
    semw = (sw0, sw1)

    pltpu.sync_copy(src_hbm.at[wid], idxs_v)
    pltpu.sync_copy(dst_hbm.at[wid], idxd_v)

    def issue_gather(i, b):
        pltpu.async_copy(p_hbm.at[idxs_v.at[i]], bufp[b], semgp[b])
        pltpu.async_copy(q_hbm.at[idxd_v.at[i]], bufq[b], semgq[b])

    def wait_gather(b):
        pltpu.make_async_copy(p_hbm.at[idxs_v.at[0]], bufp[b], semgp[b]).wait()
        pltpu.make_async_copy(q_hbm.at[idxd_v.at[0]], bufq[b], semgq[b]).wait()

    def issue_write(i, b):
        pltpu.async_copy(bufp[b], s_hbm.at[pl.ds(base + i * ECH, ECH)],
                         semw[b])

    def wait_write(b):
        pltpu.make_async_copy(bufp[b], s_hbm.at[pl.ds(0, ECH)], semw[b]).wait()

    issue_gather(0, 0)
    issue_gather(1, 1)

    def outer(k, _):
        for b in range(2):
            i = k * 2 + b
            wait_gather(b)
            p, q = bufp[b], bufq[b]

            def add_row(r, _):
                for j in range(H // 16):
                    sl = pl.ds(j * 16, 16)
                    p[r, sl] = p[r, sl] + q[r, sl]
                return _

            lax.fori_loop(0, ECH, add_row, None)
            issue_write(i, b)

            @pl.when(i + 2 < NCH_E)
            def _():
                wait_write(b)
                issue_gather(i + 2, b)

        return _

    lax.fori_loop(0, NCH_E // 2, outer, None)
    wait_write(0)
    wait_write(1)


@functools.partial(
    pl.kernel,
    out_type=jax.ShapeDtypeStruct((NC, SROWS, H), jnp.float32),
    mesh=_MESH,
    scratch_types=[
        pltpu.VMEM_SHARED((SROWS, H), jnp.float32),
        pltpu.VMEM((NCH_E, ECH), jnp.int32),
        pltpu.VMEM((ECH, H), jnp.float32),
        pltpu.VMEM((ECH, H), jnp.float32),
        pltpu.SemaphoreType.DMA,
        pltpu.SemaphoreType.DMA,
        pltpu.SemaphoreType.DMA,
        pltpu.SemaphoreType.DMA,
    ],
)
def _sc_scatter_add(msg_hbm, dst_hbm, out_hbm, acc_sh, idx_v, mb0, mb1,
                    sm0, sm1, sa0, sa1):
    c = lax.axis_index("c")
    s = lax.axis_index("s")
    wid = s * NC + c
    bufm = (mb0, mb1)
    semm = (sm0, sm1)
    sema = (sa0, sa1)

    def zrow(r, _):
        for j in range(H // 16):
            mb0[r, pl.ds(j * 16, 16)] = jnp.zeros((16,), jnp.float32)
        return _

    lax.fori_loop(0, ECH, zrow, None)

    def zchunk(i, _):
        pltpu.sync_copy(mb0, acc_sh.at[pl.ds(s * ZROWS_T + i * ECH, ECH)])
        return _

    lax.fori_loop(0, ZROWS_T // ECH, zchunk, None)
    pltpu.sync_copy(dst_hbm.at[wid], idx_v)
    plsc.subcore_barrier()

    base = wid * EW

    def issue_load(i, b):
        pltpu.async_copy(msg_hbm.at[pl.ds(base + i * ECH, ECH)], bufm[b],
                         semm[b])

    def wait_load(b):
        pltpu.make_async_copy(msg_hbm.at[pl.ds(0, ECH)], bufm[b],
                              semm[b]).wait()

    def issue_scatter(i, b):
        pltpu.async_copy(bufm[b], acc_sh.at[idx_v.at[i]], sema[b], add=True)

    def wait_scatter(b):
        pltpu.make_async_copy(bufm[b], acc_sh.at[idx_v.at[0]], sema[b]).wait()

    issue_load(0, 0)
    issue_load(1, 1)

    def outer(k, _):
        for b in range(2):
            i = k * 2 + b
            wait_load(b)
            issue_scatter(i, b)

            @pl.when(i + 2 < NCH_E)
            def _():
                wait_scatter(b)
                issue_load(i + 2, b)

        return _

    lax.fori_loop(0, NCH_E // 2, outer, None)
    wait_scatter(0)
    wait_scatter(1)
    plsc.subcore_barrier()

    pltpu.sync_copy(acc_sh.at[pl.ds(s * ZROWS_T, ZROWS_T)],
                    out_hbm.at[c, pl.ds(s * ZROWS_T, ZROWS_T)])


BN = 2000
BE = 2048


def _tc_pq_body(x_ref, a_ref, b_ref, p_ref, q_ref):
    x = x_ref[...]
    p_ref[...] = jnp.dot(x, a_ref[...], preferred_element_type=jnp.float32, precision=lax.Precision.HIGHEST)
    q_ref[...] = jnp.dot(x, b_ref[...], preferred_element_type=jnp.float32, precision=lax.Precision.HIGHEST)


def _tc_pq(x, wa, wb):
    return pl.pallas_call(
        _tc_pq_body,
        grid=(N // BN,),
        in_specs=[
            pl.BlockSpec((BN, H), lambda i: (i, 0)),
            pl.BlockSpec((H, H), lambda i: (0, 0)),
            pl.BlockSpec((H, H), lambda i: (0, 0)),
        ],
        out_specs=[
            pl.BlockSpec((BN, H), lambda i: (i, 0)),
            pl.BlockSpec((BN, H), lambda i: (i, 0)),
        ],
        out_shape=[
            jax.ShapeDtypeStruct((N, H), jnp.float32),
            jax.ShapeDtypeStruct((N, H), jnp.float32),
        ],
    )(x, wa, wb)


def _tc_edge_body(s_ref, ea_ref, wc_ref, b0_ref, w1_ref, b1_ref,
                  out_ref):
    pre = (s_ref[...]
           + jnp.dot(ea_ref[...], wc_ref[...],
                     preferred_element_type=jnp.float32, precision=lax.Precision.HIGHEST) + b0_ref[...])
    h = _silu(pre)
    out_ref[...] = _silu(
        jnp.dot(h, w1_ref[...], preferred_element_type=jnp.float32)
        + b1_ref[...])


def _tc_edge(s, ea, wc, b0, w1, b1):
    return pl.pallas_call(
        _tc_edge_body,
        grid=(EPAD // BE,),
        in_specs=[
            pl.BlockSpec((BE, H), lambda i: (i, 0)),
            pl.BlockSpec((BE, ED), lambda i: (i, 0)),
            pl.BlockSpec((ED, H), lambda i: (0, 0)),
            pl.BlockSpec((1, H), lambda i: (0, 0)),
            pl.BlockSpec((H, H), lambda i: (0, 0)),
            pl.BlockSpec((1, H), lambda i: (0, 0)),
        ],
        out_specs=pl.BlockSpec((BE, H), lambda i: (i, 0)),
        out_shape=jax.ShapeDtypeStruct((EPAD, H), jnp.float32),
    )(s, ea, wc, b0, w1, b1)


def _tc_node_body(x_ref, p0_ref, p1_ref, wn_ref, b_ref, out_ref):
    agg = p0_ref[0] + p1_ref[0]
    node_in = jnp.concatenate([x_ref[...], agg], axis=1)
    pre = jnp.dot(node_in, wn_ref[...],
                  preferred_element_type=jnp.float32) + b_ref[...]
    out_ref[...] = _silu(pre)


def _tc_node(x, parts, wn, b):
    return pl.pallas_call(
        _tc_node_body,
        grid=(N // BN,),
        in_specs=[
            pl.BlockSpec((BN, H), lambda i: (i, 0)),
            pl.BlockSpec((1, BN, H), lambda i: (0, i, 0)),
            pl.BlockSpec((1, BN, H), lambda i: (1, i, 0)),
            pl.BlockSpec((2 * H, H), lambda i: (0, 0)),
            pl.BlockSpec((1, H), lambda i: (0, 0)),
        ],
        out_specs=pl.BlockSpec((BN, H), lambda i: (i, 0)),
        out_shape=jax.ShapeDtypeStruct((N, H), jnp.float32),
    )(x, parts, parts, wn, b)


def _tc_readout_body(x_ref, batch_ref, wr0_ref, br0_ref, wr1_ref, br1_ref,
                     out_ref):
    seg = lax.broadcasted_iota(jnp.int32, (G, 1), 0)
    mask = jnp.equal(batch_ref[...], seg).astype(jnp.float32)
    counts = jnp.sum(mask, axis=1, keepdims=True)
    gsum = jnp.dot(mask, x_ref[...], preferred_element_type=jnp.float32, precision=lax.Precision.HIGHEST)
    g = gsum / jnp.maximum(counts, 1.0)
    hr = _silu(jnp.dot(g, wr0_ref[...], preferred_element_type=jnp.float32)
               + br0_ref[...])
    out_ref[...] = (jnp.dot(hr, wr1_ref[...],
                            preferred_element_type=jnp.float32) + br1_ref[...])


def _tc_readout(x, batch2d, wr0, br0, wr1, br1):
    return pl.pallas_call(
        _tc_readout_body,
        grid=(1,),
        in_specs=[
            pl.BlockSpec((N, H), lambda i: (0, 0)),
            pl.BlockSpec((1, N), lambda i: (0, 0)),
            pl.BlockSpec((H, H), lambda i: (0, 0)),
            pl.BlockSpec((1, H), lambda i: (0, 0)),
            pl.BlockSpec((H, 1), lambda i: (0, 0)),
            pl.BlockSpec((1, 1), lambda i: (0, 0)),
        ],
        out_specs=pl.BlockSpec((G, 1), lambda i: (0, 0)),
        out_shape=jax.ShapeDtypeStruct((G, 1), jnp.float32),
    )(x, batch2d, wr0, br0, wr1, br1)


def kernel(z, edge_index, edge_attr, batch, emb, We0, be0, We1, be1,
           Wn, bn, Wr0, br0, Wr1, br1):
    src = edge_index[0].astype(jnp.int32)
    dst = edge_index[1].astype(jnp.int32)

    srcp = jnp.concatenate(
        [src, jnp.zeros((EPAD - E,), jnp.int32)]).reshape(NW, NCH_E, ECH)
    dstp_g = jnp.concatenate(
        [dst, jnp.zeros((EPAD - E,), jnp.int32)]).reshape(NW, NCH_E, ECH)
    dstp_s = jnp.concatenate(
        [dst, jnp.full((EPAD - E,), N, jnp.int32)]).reshape(NW, NCH_E, ECH)
    eap = jnp.concatenate(
        [edge_attr, jnp.zeros((EPAD - E, ED), jnp.float32)], axis=0)
    zp = jnp.concatenate(
        [z.astype(jnp.int32), jnp.zeros((NZPAD - N,), jnp.int32)])

    x = _sc_gather_emb(emb, zp)[:N]

    for l in range(L):
        wa = We0[l, :H]
        wb = We0[l, H:2 * H]
        wc = We0[l, 2 * H:]
        p, q = _tc_pq(x, wa, wb)
        s = _sc_gather_edges(p, q, srcp, dstp_g)
        msg = _tc_edge(s, eap, wc, be0[l].reshape(1, H),
                       We1[l], be1[l].reshape(1, H))
        parts = _sc_scatter_add(msg, dstp_s)
        x = _tc_node(x, parts, Wn[l], bn[l].reshape(1, H))

    pred = _tc_readout(x, batch.astype(jnp.int32).reshape(1, N),
                       Wr0, br0.reshape(1, H), Wr1, br1.reshape(1, 1))
    return pred.reshape(G)


# speedup vs baseline: 2.2274x; 1.0094x over previous
import functools

import jax
import jax.numpy as jnp
from jax import lax
from jax.experimental import pallas as pl
from jax.experimental.pallas import tpu as pltpu
from jax.experimental.pallas import tpu_sc as plsc

N = 10000
E = 320000
H = 128
ED = 4
L = 4
G = 128

NC = 2
NS = 16
NW = NC * NS

ECH = 128
EW = 10240
EPAD = NW * EW
NCH_E = EW // ECH

NZW = 320
NZPAD = NW * NZW
ZCH = 80
NCH_Z = NZW // ZCH

SROWS = 10240
ZROWS_T = SROWS // NS

SECH = 128
SNCH_E = EW // SECH

_MESH = plsc.VectorSubcoreMesh(
    core_axis_name="c", subcore_axis_name="s", num_cores=NC, num_subcores=NS)


def _silu(v):
    return v * jax.nn.sigmoid(v)


@functools.partial(
    pl.kernel,
    out_type=jax.ShapeDtypeStruct((NZPAD, H), jnp.float32),
    mesh=_MESH,
    scratch_types=[
        pltpu.VMEM((ZCH,), jnp.int32),
        pltpu.VMEM((ZCH, H), jnp.float32),
        pltpu.SemaphoreType.DMA,
    ],
)
def _sc_gather_emb(table_hbm, idx_hbm, out_hbm, idx_v, rows_v, sem):
    wid = lax.axis_index("s") * NC + lax.axis_index("c")
    base = wid * NZW

    def chunk(i, _):
        off = base + i * ZCH
        pltpu.sync_copy(idx_hbm.at[pl.ds(off, ZCH)], idx_v)
        pltpu.async_copy(table_hbm.at[idx_v], rows_v, sem).wait()
        pltpu.sync_copy(rows_v, out_hbm.at[pl.ds(off, ZCH)])
        return _

    lax.fori_loop(0, NCH_Z, chunk, None)


@functools.partial(
    pl.kernel,
    out_type=jax.ShapeDtypeStruct((EPAD, H), jnp.float32),
    mesh=_MESH,
    scratch_types=[
        pltpu.VMEM((NCH_E, ECH), jnp.int32),
        pltpu.VMEM((NCH_E, ECH), jnp.int32),
        pltpu.VMEM((ECH, H), jnp.float32),
        pltpu.VMEM((ECH, H), jnp.float32),
        pltpu.VMEM((ECH, H), jnp.float32),
        pltpu.VMEM((ECH, H), jnp.float32),
        pltpu.VMEM((ECH, H), jnp.float32),
        pltpu.VMEM((ECH, H), jnp.float32),
        pltpu.SemaphoreType.DMA,
        pltpu.SemaphoreType.DMA,
        pltpu.SemaphoreType.DMA,
        pltpu.SemaphoreType.DMA,
        pltpu.SemaphoreType.DMA,
        pltpu.SemaphoreType.DMA,
    ],
)
def _sc_gather_edges(p_hbm, q_hbm, src_hbm, dst_hbm, s_hbm,
                     idxs_v, idxd_v, bp0, bp1, bq0, bq1, wb0, wb1,
                     sgp0, sgp1, sgq0, sgq1, sw0, sw1):
    wid = lax.axis_index("s") * NC + lax.axis_index("c")
    base = wid * EW
    bufp = (bp0, bp1)
    bufq = (bq0, bq1)
    wbuf = (wb0, wb1)
    semgp = (sgp0, sgp1)
    semgq = (sgq0, sgq1)
    semw = (sw0, sw1)

    pltpu.sync_copy(src_hbm.at[wid], idxs_v)
    pltpu.sync_copy(dst_hbm.at[wid], idxd_v)

    def issue_gather(i, b):
        pltpu.async_copy(p_hbm.at[idxs_v.at[i]], bufp[b], semgp[b])
        pltpu.async_copy(q_hbm.at[idxd_v.at[i]], bufq[b], semgq[b])

    def wait_gather(b):
        pltpu.make_async_copy(p_hbm.at[idxs_v.at[0]], bufp[b], semgp[b]).wait()
        pltpu.make_async_copy(q_hbm.at[idxd_v.at[0]], bufq[b], semgq[b]).wait()

    def issue_write(i, w):
        pltpu.async_copy(wbuf[w], s_hbm.at[pl.ds(base + i * ECH, ECH)],
                         semw[w])

    def wait_write(w):
        pltpu.make_async_copy(wbuf[w], s_hbm.at[pl.ds(0, ECH)], semw[w]).wait()

    issue_gather(0, 0)
    issue_gather(1, 1)

    def outer(k, _):
        for t in range(2):
            i = k * 2 + t
            b = t
            w = t
            wait_gather(b)

            @pl.when(i >= 2)
            def _():
                wait_write(w)

            p, q, o = bufp[b], bufq[b], wbuf[w]

            def add_row(r2, _):
                for rr in range(2):
                    r = r2 * 2 + rr
                    for j in range(H // 16):
                        sl = pl.ds(j * 16, 16)
                        o[r, sl] = p[r, sl] + q[r, sl]
                return _

            lax.fori_loop(0, ECH // 2, add_row, None)

            @pl.when(i + 2 < NCH_E)
            def _():
                issue_gather(i + 2, b)

            issue_write(i, w)

        return _

    lax.fori_loop(0, NCH_E // 2, outer, None)
    wait_write(0)
    wait_write(1)


@functools.partial(
    pl.kernel,
    out_type=jax.ShapeDtypeStruct((NC, SROWS, H), jnp.float32),
    mesh=_MESH,
    scratch_types=[
        pltpu.VMEM_SHARED((SROWS, H), jnp.float32),
        pltpu.VMEM((SNCH_E, SECH), jnp.int32),
        pltpu.VMEM((SECH, H), jnp.float32),
        pltpu.VMEM((SECH, H), jnp.float32),
        pltpu.SemaphoreType.DMA,
        pltpu.SemaphoreType.DMA,
        pltpu.SemaphoreType.DMA,
        pltpu.SemaphoreType.DMA,
    ],
)
def _sc_scatter_add(msg_hbm, dst_hbm, out_hbm, acc_sh, idx_v,
                    mb0, mb1, sm0, sm1, sa0, sa1):
    c = lax.axis_index("c")
    s = lax.axis_index("s")
    wid = s * NC + c
    bufm = (mb0, mb1)
    semm = (sm0, sm1)
    sema = (sa0, sa1)

    def zrow(r, _):
        for j in range(H // 16):
            mb0[r, pl.ds(j * 16, 16)] = jnp.zeros((16,), jnp.float32)
        return _

    lax.fori_loop(0, SECH, zrow, None)

    def zchunk(i, _):
        pltpu.sync_copy(mb0, acc_sh.at[pl.ds(s * ZROWS_T + i * SECH, SECH)])
        return _

    lax.fori_loop(0, ZROWS_T // SECH, zchunk, None)
    pltpu.sync_copy(dst_hbm.at[wid], idx_v)
    plsc.subcore_barrier()

    base = wid * EW

    def issue_load(i, b):
        pltpu.async_copy(msg_hbm.at[pl.ds(base + i * SECH, SECH)], bufm[b],
                         semm[b])

    def wait_load(b):
        pltpu.make_async_copy(msg_hbm.at[pl.ds(0, SECH)], bufm[b],
                              semm[b]).wait()

    def issue_scatter(i, b):
        pltpu.async_copy(bufm[b], acc_sh.at[idx_v.at[i]], sema[b], add=True)

    def wait_scatter(b):
        pltpu.make_async_copy(bufm[b], acc_sh.at[idx_v.at[0]], sema[b]).wait()

    issue_load(0, 0)
    issue_load(1, 1)

    def outer(k, _):
        for b in range(2):
            i = k * 2 + b
            wait_load(b)
            issue_scatter(i, b)

            @pl.when(i + 2 < SNCH_E)
            def _():
                wait_scatter(b)
                issue_load(i + 2, b)

        return _

    lax.fori_loop(0, SNCH_E // 2, outer, None)
    wait_scatter(0)
    wait_scatter(1)
    plsc.subcore_barrier()

    pltpu.sync_copy(acc_sh.at[pl.ds(s * ZROWS_T, ZROWS_T)],
                    out_hbm.at[c, pl.ds(s * ZROWS_T, ZROWS_T)])


BN = 2000
BE = 2048


def _tc_pq_body(x_ref, a_ref, b_ref, p_ref, q_ref):
    x = x_ref[...]
    p_ref[...] = jnp.dot(x, a_ref[...], preferred_element_type=jnp.float32, precision=lax.Precision.HIGHEST)
    q_ref[...] = jnp.dot(x, b_ref[...], preferred_element_type=jnp.float32, precision=lax.Precision.HIGHEST)


def _tc_pq(x, wa, wb):
    return pl.pallas_call(
        _tc_pq_body,
        grid=(N // BN,),
        in_specs=[
            pl.BlockSpec((BN, H), lambda i: (i, 0)),
            pl.BlockSpec((H, H), lambda i: (0, 0)),
            pl.BlockSpec((H, H), lambda i: (0, 0)),
        ],
        out_specs=[
            pl.BlockSpec((BN, H), lambda i: (i, 0)),
            pl.BlockSpec((BN, H), lambda i: (i, 0)),
        ],
        out_shape=[
            jax.ShapeDtypeStruct((N, H), jnp.float32),
            jax.ShapeDtypeStruct((N, H), jnp.float32),
        ],
    )(x, wa, wb)


def _tc_edge_body(s_ref, ea_ref, wc_ref, b0_ref, w1_ref, b1_ref,
                  out_ref):
    pre = (s_ref[...]
           + jnp.dot(ea_ref[...], wc_ref[...],
                     preferred_element_type=jnp.float32, precision=lax.Precision.HIGHEST) + b0_ref[...])
    h = _silu(pre)
    out_ref[...] = _silu(
        jnp.dot(h, w1_ref[...], preferred_element_type=jnp.float32)
        + b1_ref[...])


def _tc_edge(s, ea, wc, b0, w1, b1):
    return pl.pallas_call(
        _tc_edge_body,
        grid=(EPAD // BE,),
        in_specs=[
            pl.BlockSpec((BE, H), lambda i: (i, 0)),
            pl.BlockSpec((BE, ED), lambda i: (i, 0)),
            pl.BlockSpec((ED, H), lambda i: (0, 0)),
            pl.BlockSpec((1, H), lambda i: (0, 0)),
            pl.BlockSpec((H, H), lambda i: (0, 0)),
            pl.BlockSpec((1, H), lambda i: (0, 0)),
        ],
        out_specs=pl.BlockSpec((BE, H), lambda i: (i, 0)),
        out_shape=jax.ShapeDtypeStruct((EPAD, H), jnp.float32),
    )(s, ea, wc, b0, w1, b1)


def _tc_node_body(x_ref, p0_ref, p1_ref, wn_ref, b_ref, out_ref):
    agg = p0_ref[0] + p1_ref[0]
    node_in = jnp.concatenate([x_ref[...], agg], axis=1)
    pre = jnp.dot(node_in, wn_ref[...],
                  preferred_element_type=jnp.float32) + b_ref[...]
    out_ref[...] = _silu(pre)


def _tc_node(x, parts, wn, b):
    return pl.pallas_call(
        _tc_node_body,
        grid=(N // BN,),
        in_specs=[
            pl.BlockSpec((BN, H), lambda i: (i, 0)),
            pl.BlockSpec((1, BN, H), lambda i: (0, i, 0)),
            pl.BlockSpec((1, BN, H), lambda i: (1, i, 0)),
            pl.BlockSpec((2 * H, H), lambda i: (0, 0)),
            pl.BlockSpec((1, H), lambda i: (0, 0)),
        ],
        out_specs=pl.BlockSpec((BN, H), lambda i: (i, 0)),
        out_shape=jax.ShapeDtypeStruct((N, H), jnp.float32),
    )(x, parts, parts, wn, b)


def _tc_readout_body(x_ref, batch_ref, wr0_ref, br0_ref, wr1_ref, br1_ref,
                     out_ref):
    seg = lax.broadcasted_iota(jnp.int32, (G, 1), 0)
    mask = jnp.equal(batch_ref[...], seg).astype(jnp.float32)
    counts = jnp.sum(mask, axis=1, keepdims=True)
    gsum = jnp.dot(mask, x_ref[...], preferred_element_type=jnp.float32, precision=lax.Precision.HIGHEST)
    g = gsum / jnp.maximum(counts, 1.0)
    hr = _silu(jnp.dot(g, wr0_ref[...], preferred_element_type=jnp.float32)
               + br0_ref[...])
    out_ref[...] = (jnp.dot(hr, wr1_ref[...],
                            preferred_element_type=jnp.float32) + br1_ref[...])


def _tc_readout(x, batch2d, wr0, br0, wr1, br1):
    return pl.pallas_call(
        _tc_readout_body,
        grid=(1,),
        in_specs=[
            pl.BlockSpec((N, H), lambda i: (0, 0)),
            pl.BlockSpec((1, N), lambda i: (0, 0)),
            pl.BlockSpec((H, H), lambda i: (0, 0)),
            pl.BlockSpec((1, H), lambda i: (0, 0)),
            pl.BlockSpec((H, 1), lambda i: (0, 0)),
            pl.BlockSpec((1, 1), lambda i: (0, 0)),
        ],
        out_specs=pl.BlockSpec((G, 1), lambda i: (0, 0)),
        out_shape=jax.ShapeDtypeStruct((G, 1), jnp.float32),
    )(x, batch2d, wr0, br0, wr1, br1)


def kernel(z, edge_index, edge_attr, batch, emb, We0, be0, We1, be1,
           Wn, bn, Wr0, br0, Wr1, br1):
    src = edge_index[0].astype(jnp.int32)
    dst = edge_index[1].astype(jnp.int32)

    srcp = jnp.concatenate(
        [src, jnp.zeros((EPAD - E,), jnp.int32)]).reshape(NW, NCH_E, ECH)
    dstp_g = jnp.concatenate(
        [dst, jnp.zeros((EPAD - E,), jnp.int32)]).reshape(NW, NCH_E, ECH)
    dstp_s = jnp.concatenate(
        [dst, jnp.full((EPAD - E,), N, jnp.int32)]).reshape(NW, SNCH_E, SECH)
    eap = jnp.concatenate(
        [edge_attr, jnp.zeros((EPAD - E, ED), jnp.float32)], axis=0)
    zp = jnp.concatenate(
        [z.astype(jnp.int32), jnp.zeros((NZPAD - N,), jnp.int32)])

    x = _sc_gather_emb(emb, zp)[:N]

    for l in range(L):
        wa = We0[l, :H]
        wb = We0[l, H:2 * H]
        wc = We0[l, 2 * H:]
        p, q = _tc_pq(x, wa, wb)
        s = _sc_gather_edges(p, q, srcp, dstp_g)
        msg = _tc_edge(s, eap, wc, be0[l].reshape(1, H),
                       We1[l], be1[l].reshape(1, H))
        parts = _sc_scatter_add(msg, dstp_s)
        x = _tc_node(x, parts, Wn[l], bn[l].reshape(1, H))

    pred = _tc_readout(x, batch.astype(jnp.int32).reshape(1, N),
                       Wr0, br0.reshape(1, H), Wr1, br1.reshape(1, 1))
    return pred.reshape(G)
